# baseline (speedup 1.0000x reference)
import functools

import jax
import jax.numpy as jnp
from jax import lax
from jax.experimental import pallas as pl
from jax.experimental.pallas import tpu as pltpu
from jax.experimental.pallas import tpu_sc as plsc

VOCAB = 100000
DIM = 768
CNTX = 2048
HEADS = 12
EXPERTS = 8
HID = 3072
NCLS = 10
DH = DIM // HEADS
CAP = max(1, int(CNTX / EXPERTS * 1.25))
EPS = 1e-6
F32 = jnp.float32

SC_CORES = 2
SC_SUBCORES = 16
SC_WORKERS = SC_CORES * SC_SUBCORES
TOK_PER_W = CNTX // SC_WORKERS


def _embed_gather(ids, emb):
    mesh = plsc.VectorSubcoreMesh(core_axis_name="c", subcore_axis_name="s")

    @functools.partial(
        pl.kernel,
        out_type=jax.ShapeDtypeStruct((CNTX, DIM), F32),
        mesh=mesh,
        scratch_types=[
            pltpu.VMEM((TOK_PER_W,), jnp.int32),
            pltpu.VMEM((TOK_PER_W, DIM), F32),
            pltpu.SemaphoreType.DMA,
        ],
    )
    def k(ids_hbm, emb_hbm, out_hbm, idx_v, rows_v, sem):
        wid = lax.axis_index("s") * SC_CORES + lax.axis_index("c")
        base = wid * TOK_PER_W
        pltpu.sync_copy(ids_hbm.at[pl.ds(base, TOK_PER_W)], idx_v)
        pltpu.async_copy(emb_hbm.at[idx_v], rows_v, sem).wait()
        pltpu.sync_copy(rows_v, out_hbm.at[pl.ds(base, TOK_PER_W)])

    return k(ids, emb)


def _pre_attn(a, b, nw, wqkv, bqkv):
    def body(a_ref, b_ref, nw_ref, wq_ref, bq_ref, h0_ref, qkv_ref):
        h = a_ref[...] + b_ref[...]
        h0_ref[...] = h
        ms = jnp.mean(h * h, axis=-1, keepdims=True)
        xn = h * lax.rsqrt(ms + EPS) * nw_ref[...]
        qkv_ref[...] = (
            jnp.dot(xn, wq_ref[...], preferred_element_type=F32) + bq_ref[...]
        )

    return pl.pallas_call(
        body,
        out_shape=(
            jax.ShapeDtypeStruct((CNTX, DIM), F32),
            jax.ShapeDtypeStruct((CNTX, 3 * DIM), F32),
        ),
    )(a, b, nw, wqkv, bqkv)


def _attn(qkv):
    hb = 2 * DH
    g = HEADS // 2

    ck = CNTX // 2

    def body(q_ref, k_ref, v_ref, o_ref):
        outs = []
        for j in range(2):
            sl = slice(j * DH, (j + 1) * DH)
            q = q_ref[:, sl]
            k = k_ref[:, sl]
            s = lax.dot_general(
                q, k, (((1,), (1,)), ((), ())), preferred_element_type=F32
            ) * (float(DH) ** -0.5)
            v = v_ref[:, sl]
            s0, s1 = s[:, :ck], s[:, ck:]
            v0, v1 = v[:ck, :], v[ck:, :]
            m1 = jnp.max(s0, axis=-1, keepdims=True)
            p0 = jnp.exp(s0 - m1)
            l1 = jnp.sum(p0, axis=-1, keepdims=True)
            acc = jnp.dot(p0, v0, preferred_element_type=F32)
            m2 = jnp.maximum(m1, jnp.max(s1, axis=-1, keepdims=True))
            alpha = jnp.exp(m1 - m2)
            p1 = jnp.exp(s1 - m2)
            l2 = alpha * l1 + jnp.sum(p1, axis=-1, keepdims=True)
            acc = alpha * acc + jnp.dot(p1, v1, preferred_element_type=F32)
            outs.append(acc / l2)
        o_ref[...] = jnp.concatenate(outs, axis=1)

    return pl.pallas_call(
        body,
        grid=(g,),
        in_specs=[
            pl.BlockSpec((CNTX, hb), lambda h: (0, h)),
            pl.BlockSpec((CNTX, hb), lambda h: (0, g + h)),
            pl.BlockSpec((CNTX, hb), lambda h: (0, 2 * g + h)),
        ],
        out_specs=pl.BlockSpec((CNTX, hb), lambda h: (0, h)),
        out_shape=jax.ShapeDtypeStruct((CNTX, DIM), F32),
    )(qkv, qkv, qkv)


def _post_attn(h0, ao, wo, bo, nw):
    def body(h_ref, ao_ref, wo_ref, bo_ref, nw_ref, h1_ref, xn_ref):
        h1 = (
            h_ref[...]
            + jnp.dot(ao_ref[...], wo_ref[...], preferred_element_type=F32)
            + bo_ref[...]
        )
        h1_ref[...] = h1
        ms = jnp.mean(h1 * h1, axis=-1, keepdims=True)
        xn_ref[...] = h1 * lax.rsqrt(ms + EPS) * nw_ref[...]

    return pl.pallas_call(
        body,
        out_shape=(
            jax.ShapeDtypeStruct((CNTX, DIM), F32),
            jax.ShapeDtypeStruct((CNTX, DIM), F32),
        ),
    )(h0, ao, wo, bo, nw)


def _router(xn, wr, br):
    def body(xn_ref, wr_ref, br_ref, idx_ref, kg_ref):
        logits = (
            jnp.dot(xn_ref[...], wr_ref[...], preferred_element_type=F32)
            + br_ref[...]
        )
        m = jnp.max(logits, axis=-1, keepdims=True)
        ex = jnp.exp(logits - m)
        probs = ex / jnp.sum(ex, axis=-1, keepdims=True)
        gate = jnp.max(probs, axis=-1, keepdims=True)
        cidx = lax.broadcasted_iota(jnp.int32, (CNTX, EXPERTS), 1)
        expert = jnp.min(
            jnp.where(probs == gate, cidx, EXPERTS), axis=-1, keepdims=True
        )
        oh = (cidx == expert).astype(jnp.bfloat16)
        r = lax.broadcasted_iota(jnp.int32, (CNTX, CNTX), 0)
        c = lax.broadcasted_iota(jnp.int32, (CNTX, CNTX), 1)
        tri = (r >= c).astype(jnp.bfloat16)
        cnt = jnp.dot(tri, oh, preferred_element_type=F32)
        posn = jnp.sum((cnt - 1.0) * oh.astype(F32), axis=-1, keepdims=True)
        keep = posn < CAP
        idx_ref[...] = jnp.where(
            keep, expert * CAP + posn.astype(jnp.int32), EXPERTS * CAP
        )
        kg_ref[...] = jnp.where(keep, gate, 0.0)

    return pl.pallas_call(
        body,
        out_shape=(
            jax.ShapeDtypeStruct((CNTX, 1), jnp.int32),
            jax.ShapeDtypeStruct((CNTX, 1), F32),
        ),
    )(xn, wr, br)


HIDB = HID // 2


def _moe(xn, idx, kg, w1, b1, w2, b2):
    def body(
        xn_ref, idx_ref, kg_ref, w1_ref, b1_ref, w2_ref, b2_ref, out_ref,
        buf_s, yacc,
    ):
        e = pl.program_id(0)
        kk = pl.program_id(1)

        @pl.when(kk == 0)
        def _():
            slot = lax.broadcasted_iota(jnp.int32, (CNTX, CAP), 1)
            disp = (idx_ref[...] == slot + e * CAP).astype(F32)
            buf_s[...] = lax.dot_general(
                disp, xn_ref[...], (((0,), (0,)), ((), ())),
                preferred_element_type=F32,
                precision=lax.Precision.HIGHEST,
            )

        h = jnp.maximum(
            jnp.dot(buf_s[...], w1_ref[0], preferred_element_type=F32)
            + b1_ref[0],
            0.0,
        )
        yp = jnp.dot(h, w2_ref[0], preferred_element_type=F32)

        @pl.when(kk == 0)
        def _():
            yacc[...] = yp + b2_ref[0]

        @pl.when(kk == 1)
        def _():
            y = yacc[...] + yp
            slot = lax.broadcasted_iota(jnp.int32, (CNTX, CAP), 1)
            comb = (idx_ref[...] == slot + e * CAP).astype(F32)
            contrib = (
                jnp.dot(
                    comb, y, preferred_element_type=F32,
                    precision=lax.Precision.HIGHEST,
                )
                * kg_ref[...]
            )

            @pl.when(e == 0)
            def _():
                out_ref[...] = contrib

            @pl.when(e > 0)
            def _():
                out_ref[...] += contrib

    return pl.pallas_call(
        body,
        grid=(EXPERTS, 2),
        in_specs=[
            pl.BlockSpec((CNTX, DIM), lambda e, k: (0, 0)),
            pl.BlockSpec((CNTX, 1), lambda e, k: (0, 0)),
            pl.BlockSpec((CNTX, 1), lambda e, k: (0, 0)),
            pl.BlockSpec((1, DIM, HIDB), lambda e, k: (e, 0, k)),
            pl.BlockSpec((1, 1, HIDB), lambda e, k: (e, 0, k)),
            pl.BlockSpec((1, HIDB, DIM), lambda e, k: (e, k, 0)),
            pl.BlockSpec((1, 1, DIM), lambda e, k: (e, 0, 0)),
        ],
        out_specs=pl.BlockSpec((CNTX, DIM), lambda e, k: (0, 0)),
        out_shape=jax.ShapeDtypeStruct((CNTX, DIM), F32),
        scratch_shapes=[
            pltpu.VMEM((CAP, DIM), F32),
            pltpu.VMEM((CAP, DIM), F32),
        ],
    )(xn, idx, kg, w1, b1, w2, b2)


KBLK = 16384
NKB = DIM * CNTX // KBLK


def _classifier(af, bf, wout, bout):
    def body(a_ref, b_ref, w_ref, bo_ref, o_ref):
        i = pl.program_id(0)
        part = jnp.dot(
            a_ref[...] + b_ref[...], w_ref[...], preferred_element_type=F32
        )

        @pl.when(i == 0)
        def _():
            o_ref[...] = part + bo_ref[...]

        @pl.when(i > 0)
        def _():
            o_ref[...] += part

    return pl.pallas_call(
        body,
        grid=(NKB,),
        in_specs=[
            pl.BlockSpec((1, KBLK), lambda i: (0, i)),
            pl.BlockSpec((1, KBLK), lambda i: (0, i)),
            pl.BlockSpec((KBLK, NCLS), lambda i: (i, 0)),
            pl.BlockSpec((NCLS,), lambda i: (0,)),
        ],
        out_specs=pl.BlockSpec((1, NCLS), lambda i: (0, 0)),
        out_shape=jax.ShapeDtypeStruct((1, NCLS), F32),
    )(af, bf, wout, bout)


def kernel(params, x):
    p = params
    ids = x.reshape(-1).astype(jnp.int32)
    g = _embed_gather(ids, p["embed"])
    a, b = g, p["pos"]
    for lp in p["layers"]:
        h0, qkv = _pre_attn(a, b, lp["attn_norm_w"], lp["Wqkv"], lp["bqkv"])
        ao = _attn(qkv)
        h1, xn = _post_attn(h0, ao, lp["Wo"], lp["bo"], lp["ffn_norm_w"])
        idx, kg = _router(xn, lp["Wr"], lp["br"])
        msum = _moe(
            xn, idx, kg, lp["W1"],
            lp["b1"].reshape(EXPERTS, 1, HID),
            lp["W2"],
            lp["b2"].reshape(EXPERTS, 1, DIM),
        )
        a, b = h1, msum
    return _classifier(
        a.reshape(1, -1), b.reshape(1, -1), p["Wout"], p["bout"]
    )
